# baseline (device time: 1211965 ns/iter reference)
import jax
import jax.numpy as jnp
from jax import lax
from jax.experimental import pallas as pl
from jax.experimental.pallas import tpu as pltpu

N_DEV = 4
M, K, N = 4096, 4096, 8192
KS = K // N_DEV
CH = M // N_DEV
NH = N // 2
B = 8
SUB = CH // B
GSUB = 256
GB = CH // GSUB
PW = 1024
NP = N // PW

_GELU_C = 0.7978845608028654


def _gelu(y):
    return 0.5 * y * (1.0 + jnp.tanh(_GELU_C * (y + 0.044715 * y * y * y)))


def kernel(x, w_mat):
    def body(x_ref, w_ref, out_ref, part_ref, recv_ref, acc_ref,
             vw, vx, vo, va, vb, gemm_sems, copy_sems,
             rs_send_sems, rs_recv_sems, ag_send_sems, ag_recv_sems):
        d = lax.axis_index("i")
        left = (d - 1) % N_DEV
        right = (d + 1) % N_DEV

        barrier = pltpu.get_barrier_semaphore()
        for nbr in (left, right):
            pl.semaphore_signal(barrier, inc=1, device_id=(nbr,),
                                device_id_type=pl.DeviceIdType.MESH)
        pl.semaphore_wait(barrier, 2)

        def cols(r):
            return pl.ds(r * NH, NH)

        def sub_rows(c, k):
            return pl.ds(c * CH + k * SUB, SUB)

        def subs(k):
            return pl.ds(k * SUB, SUB)

        def chunk_send(r, s):
            return ((d - s) if r == 0 else (d + s)) % N_DEV

        def owner(r):
            return ((d + 1) if r == 0 else (d - 1)) % N_DEV

        def peer(r):
            return right if r == 0 else left

        end_waits = []

        def rs_wait_recv(r, s, k):
            rd = pltpu.make_async_remote_copy(
                src_ref=recv_ref.at[r, s, subs(k), :],
                dst_ref=recv_ref.at[r, s, subs(k), :],
                send_sem=rs_send_sems.at[r, s, k],
                recv_sem=rs_recv_sems.at[r, s, k],
                device_id=(peer(r),),
                device_id_type=pl.DeviceIdType.MESH,
            )
            rd.wait_recv()

        def s0_sends(r):
            c = chunk_send(r, 0)
            for k in range(B):
                rd = pltpu.make_async_remote_copy(
                    src_ref=part_ref.at[sub_rows(c, k), cols(r)],
                    dst_ref=recv_ref.at[r, 0, subs(k), :],
                    send_sem=rs_send_sems.at[r, 0, k],
                    recv_sem=rs_recv_sems.at[r, 0, k],
                    device_id=(peer(r),),
                    device_id_type=pl.DeviceIdType.MESH,
                )
                rd.start()
                end_waits.append(rd)

        chunk_ids = [d, (d - 1) % N_DEV, (d + 1) % N_DEV, (d + 2) % N_DEV]

        sched = [(ci, j) for ci in range(4) for j in range(NP)]

        vw_pending = [None, None]

        def start_w(i):
            _, j = sched[i]
            q = i % 2
            cp = pltpu.make_async_copy(
                w_ref.at[:, pl.ds(j * PW, PW)], vw.at[q], gemm_sems.at[0, q])
            cp.start()
            vw_pending[q] = cp

        vx_pending = [None]

        def start_x(ci):
            cp = pltpu.make_async_copy(
                x_ref.at[pl.ds(chunk_ids[ci] * CH, CH), :], vx,
                gemm_sems.at[1, 0])
            cp.start()
            vx_pending[0] = cp

        vo_pending = [None, None]
        gemm_flush = []
        keep_alive = []
        waited = set()

        def wait_once(cp):
            if cp is not None and id(cp) not in waited:
                keep_alive.append(cp)
                waited.add(id(cp))
                cp.wait()

        start_w(0)
        start_w(1)
        start_x(0)
        vx_pending[0].wait()

        for i, (ci, j) in enumerate(sched):
            q = i % 2
            vw_pending[q].wait()
            vw_pending[q] = None
            c = chunk_ids[ci]
            for k in range(GB):
                qo = k % 2
                wait_once(vo_pending[qo])
                vo[qo] = jnp.dot(vx[pl.ds(k * GSUB, GSUB), :], vw[q],
                                 preferred_element_type=jnp.float32)
                cp = pltpu.make_async_copy(
                    vo.at[qo],
                    part_ref.at[pl.ds(c * CH + k * GSUB, GSUB), pl.ds(j * PW, PW)],
                    gemm_sems.at[2, qo])
                cp.start()
                vo_pending[qo] = cp
                gemm_flush.append(cp)
            if i + 2 < len(sched):
                start_w(i + 2)
            if j == NP - 1 and ci + 1 < 4:
                start_x(ci + 1)
            if ci == 0 and j == NP // 2 - 1:
                for cp in gemm_flush:
                    wait_once(cp)
                gemm_flush.clear()
                s0_sends(0)
            if ci == 0 and j == NP - 1:
                for cp in gemm_flush:
                    wait_once(cp)
                gemm_flush.clear()
                s0_sends(1)
                vx_pending[0].wait()
            elif j == NP - 1 and ci + 1 < 4:
                vx_pending[0].wait()
        for cp in gemm_flush:
            wait_once(cp)

        for s in range(1, N_DEV - 1):
            for k in range(B):
                for r in (0, 1):
                    c = chunk_send(r, s)
                    rs_wait_recv(r, s - 1, k)
                    cp_a = pltpu.make_async_copy(
                        recv_ref.at[r, s - 1, subs(k), :],
                        va.at[r], copy_sems.at[r, 0])
                    cp_b = pltpu.make_async_copy(
                        part_ref.at[sub_rows(c, k), cols(r)],
                        vb.at[r], copy_sems.at[r, 1])
                    cp_a.start()
                    cp_b.start()
                    cp_a.wait()
                    cp_b.wait()
                    va[r] = va[r] + vb[r]
                    cp_o = pltpu.make_async_copy(
                        va.at[r], acc_ref.at[r, s - 1, subs(k), :],
                        copy_sems.at[r, 2])
                    cp_o.start()
                    cp_o.wait()
                    rd = pltpu.make_async_remote_copy(
                        src_ref=acc_ref.at[r, s - 1, subs(k), :],
                        dst_ref=recv_ref.at[r, s, subs(k), :],
                        send_sem=rs_send_sems.at[r, s, k],
                        recv_sem=rs_recv_sems.at[r, s, k],
                        device_id=(peer(r),),
                        device_id_type=pl.DeviceIdType.MESH,
                    )
                    rd.start()
                    end_waits.append(rd)

        for k in range(B):
            for r in (0, 1):
                o = owner(r)
                rs_wait_recv(r, N_DEV - 2, k)
                cp_a = pltpu.make_async_copy(
                    recv_ref.at[r, N_DEV - 2, subs(k), :],
                    va.at[r], copy_sems.at[r, 0])
                cp_b = pltpu.make_async_copy(
                    part_ref.at[sub_rows(o, k), cols(r)],
                    vb.at[r], copy_sems.at[r, 1])
                cp_a.start()
                cp_b.start()
                cp_a.wait()
                cp_b.wait()
                va[r] = _gelu(va[r] + vb[r])
                cp_o = pltpu.make_async_copy(
                    va.at[r], out_ref.at[sub_rows(o, k), cols(r)],
                    copy_sems.at[r, 2])
                cp_o.start()
                cp_o.wait()
                rd = pltpu.make_async_remote_copy(
                    src_ref=out_ref.at[sub_rows(o, k), cols(r)],
                    dst_ref=out_ref.at[sub_rows(o, k), cols(r)],
                    send_sem=ag_send_sems.at[r, 0, k],
                    recv_sem=ag_recv_sems.at[r, 0, k],
                    device_id=(peer(r),),
                    device_id_type=pl.DeviceIdType.MESH,
                )
                rd.start()
                end_waits.append(rd)

        for s in range(1, N_DEV - 1):
            for k in range(B):
                for r in (0, 1):
                    c_prev = ((d - s + 1) if r == 0 else (d + s - 1)) % N_DEV
                    rd = pltpu.make_async_remote_copy(
                        src_ref=out_ref.at[sub_rows(c_prev, k), cols(r)],
                        dst_ref=out_ref.at[sub_rows(c_prev, k), cols(r)],
                        send_sem=ag_send_sems.at[r, s - 1, k],
                        recv_sem=ag_recv_sems.at[r, s - 1, k],
                        device_id=(peer(r),),
                        device_id_type=pl.DeviceIdType.MESH,
                    )
                    rd.wait_recv()
                    fw = pltpu.make_async_remote_copy(
                        src_ref=out_ref.at[sub_rows(c_prev, k), cols(r)],
                        dst_ref=out_ref.at[sub_rows(c_prev, k), cols(r)],
                        send_sem=ag_send_sems.at[r, s, k],
                        recv_sem=ag_recv_sems.at[r, s, k],
                        device_id=(peer(r),),
                        device_id_type=pl.DeviceIdType.MESH,
                    )
                    fw.start()
                    end_waits.append(fw)

        for k in range(B):
            for r in (0, 1):
                c_last = ((d - N_DEV + 2) if r == 0 else (d + N_DEV - 2)) % N_DEV
                rd = pltpu.make_async_remote_copy(
                    src_ref=out_ref.at[sub_rows(c_last, k), cols(r)],
                    dst_ref=out_ref.at[sub_rows(c_last, k), cols(r)],
                    send_sem=ag_send_sems.at[r, N_DEV - 2, k],
                    recv_sem=ag_recv_sems.at[r, N_DEV - 2, k],
                    device_id=(peer(r),),
                    device_id_type=pl.DeviceIdType.MESH,
                )
                rd.wait_recv()
        for rd in end_waits:
            rd.wait_send()

    out = pl.pallas_call(
        body,
        out_shape=[
            jax.ShapeDtypeStruct((M, N), jnp.float32),
            jax.ShapeDtypeStruct((M, N), jnp.float32),
            jax.ShapeDtypeStruct((2, N_DEV - 1, CH, NH), jnp.float32),
            jax.ShapeDtypeStruct((2, N_DEV - 2, CH, NH), jnp.float32),
        ],
        in_specs=[pl.BlockSpec(memory_space=pl.ANY),
                  pl.BlockSpec(memory_space=pl.ANY)],
        out_specs=[pl.BlockSpec(memory_space=pl.ANY)] * 4,
        scratch_shapes=[
            pltpu.VMEM((2, KS, PW), jnp.float32),
            pltpu.VMEM((CH, KS), jnp.float32),
            pltpu.VMEM((2, GSUB, PW), jnp.float32),
            pltpu.VMEM((2, SUB, NH), jnp.float32),
            pltpu.VMEM((2, SUB, NH), jnp.float32),
            pltpu.SemaphoreType.DMA((3, 2)),
            pltpu.SemaphoreType.DMA((2, 3)),
            pltpu.SemaphoreType.DMA((2, N_DEV - 1, B)),
            pltpu.SemaphoreType.DMA((2, N_DEV - 1, B)),
            pltpu.SemaphoreType.DMA((2, N_DEV - 1, B)),
            pltpu.SemaphoreType.DMA((2, N_DEV - 1, B)),
        ],
        compiler_params=pltpu.CompilerParams(collective_id=0),
    )(x, w_mat)[0]
    return out


# device time: 1210930 ns/iter; 1.0009x vs baseline; 1.0009x over previous
import jax
import jax.numpy as jnp
from jax import lax
from jax.experimental import pallas as pl
from jax.experimental.pallas import tpu as pltpu

N_DEV = 4
M, K, N = 4096, 4096, 8192
KS = K // N_DEV
CH = M // N_DEV
NH = N // 2
B = 4
SUB = CH // B
PW = 1024
NP = N // PW

_GELU_C = 0.7978845608028654


def _gelu(y):
    return 0.5 * y * (1.0 + jnp.tanh(_GELU_C * (y + 0.044715 * y * y * y)))


def kernel(x, w_mat):
    def body(x_ref, w_ref, out_ref, part_ref, recv_ref, acc_ref,
             vw, vx, vo, va, vb, gemm_sems, copy_sems,
             rs_send_sems, rs_recv_sems, ag_send_sems, ag_recv_sems):
        d = lax.axis_index("i")
        left = (d - 1) % N_DEV
        right = (d + 1) % N_DEV

        barrier = pltpu.get_barrier_semaphore()
        for nbr in (left, right):
            pl.semaphore_signal(barrier, inc=1, device_id=(nbr,),
                                device_id_type=pl.DeviceIdType.MESH)
        pl.semaphore_wait(barrier, 2)

        def cols(r):
            return pl.ds(r * NH, NH)

        def sub_rows(c, k):
            return pl.ds(c * CH + k * SUB, SUB)

        def subs(k):
            return pl.ds(k * SUB, SUB)

        def chunk_send(r, s):
            return ((d - s) if r == 0 else (d + s)) % N_DEV

        def owner(r):
            return ((d + 1) if r == 0 else (d - 1)) % N_DEV

        def peer(r):
            return right if r == 0 else left

        end_waits = []

        def rs_wait_recv(r, s, k):
            rd = pltpu.make_async_remote_copy(
                src_ref=recv_ref.at[r, s, subs(k), :],
                dst_ref=recv_ref.at[r, s, subs(k), :],
                send_sem=rs_send_sems.at[r, s, k],
                recv_sem=rs_recv_sems.at[r, s, k],
                device_id=(peer(r),),
                device_id_type=pl.DeviceIdType.MESH,
            )
            rd.wait_recv()

        def s0_sends(r):
            c = chunk_send(r, 0)
            for k in range(B):
                rd = pltpu.make_async_remote_copy(
                    src_ref=part_ref.at[sub_rows(c, k), cols(r)],
                    dst_ref=recv_ref.at[r, 0, subs(k), :],
                    send_sem=rs_send_sems.at[r, 0, k],
                    recv_sem=rs_recv_sems.at[r, 0, k],
                    device_id=(peer(r),),
                    device_id_type=pl.DeviceIdType.MESH,
                )
                rd.start()
                end_waits.append(rd)

        chunk_ids = [d, (d - 1) % N_DEV, (d + 1) % N_DEV, (d + 2) % N_DEV]

        sched = [(ci, j) for ci in range(4) for j in range(NP)]

        vw_pending = [None, None]

        def start_w(i):
            _, j = sched[i]
            q = i % 2
            cp = pltpu.make_async_copy(
                w_ref.at[:, pl.ds(j * PW, PW)], vw.at[q], gemm_sems.at[0, q])
            cp.start()
            vw_pending[q] = cp

        vx_pending = [None]

        def start_x(ci):
            cp = pltpu.make_async_copy(
                x_ref.at[pl.ds(chunk_ids[ci] * CH, CH), :], vx,
                gemm_sems.at[1, 0])
            cp.start()
            vx_pending[0] = cp

        vo_pending = [None, None]
        gemm_flush = []
        keep_alive = []
        waited = set()

        def wait_once(cp):
            if cp is not None and id(cp) not in waited:
                keep_alive.append(cp)
                waited.add(id(cp))
                cp.wait()

        start_w(0)
        start_w(1)
        start_x(0)
        vx_pending[0].wait()

        for i, (ci, j) in enumerate(sched):
            q = i % 2
            vw_pending[q].wait()
            vw_pending[q] = None
            c = chunk_ids[ci]
            for k in range(B):
                qo = k % 2
                wait_once(vo_pending[qo])
                vo[qo] = jnp.dot(vx[pl.ds(k * SUB, SUB), :], vw[q],
                                 preferred_element_type=jnp.float32)
                cp = pltpu.make_async_copy(
                    vo.at[qo],
                    part_ref.at[sub_rows(c, k), pl.ds(j * PW, PW)],
                    gemm_sems.at[2, qo])
                cp.start()
                vo_pending[qo] = cp
                gemm_flush.append(cp)
            if i + 2 < len(sched):
                start_w(i + 2)
            if j == NP - 1 and ci + 1 < 4:
                start_x(ci + 1)
            if ci == 0 and j == NP // 2 - 1:
                for cp in gemm_flush:
                    wait_once(cp)
                gemm_flush.clear()
                s0_sends(0)
            if ci == 0 and j == NP - 1:
                for cp in gemm_flush:
                    wait_once(cp)
                gemm_flush.clear()
                s0_sends(1)
                vx_pending[0].wait()
            elif j == NP - 1 and ci + 1 < 4:
                vx_pending[0].wait()
        for cp in gemm_flush:
            wait_once(cp)

        for s in range(1, N_DEV - 1):
            for k in range(B):
                for r in (0, 1):
                    c = chunk_send(r, s)
                    rs_wait_recv(r, s - 1, k)
                    cp_a = pltpu.make_async_copy(
                        recv_ref.at[r, s - 1, subs(k), :],
                        va.at[r], copy_sems.at[r, 0])
                    cp_b = pltpu.make_async_copy(
                        part_ref.at[sub_rows(c, k), cols(r)],
                        vb.at[r], copy_sems.at[r, 1])
                    cp_a.start()
                    cp_b.start()
                    cp_a.wait()
                    cp_b.wait()
                    va[r] = va[r] + vb[r]
                    cp_o = pltpu.make_async_copy(
                        va.at[r], acc_ref.at[r, s - 1, subs(k), :],
                        copy_sems.at[r, 2])
                    cp_o.start()
                    cp_o.wait()
                    rd = pltpu.make_async_remote_copy(
                        src_ref=acc_ref.at[r, s - 1, subs(k), :],
                        dst_ref=recv_ref.at[r, s, subs(k), :],
                        send_sem=rs_send_sems.at[r, s, k],
                        recv_sem=rs_recv_sems.at[r, s, k],
                        device_id=(peer(r),),
                        device_id_type=pl.DeviceIdType.MESH,
                    )
                    rd.start()
                    end_waits.append(rd)

        for k in range(B):
            for r in (0, 1):
                o = owner(r)
                rs_wait_recv(r, N_DEV - 2, k)
                cp_a = pltpu.make_async_copy(
                    recv_ref.at[r, N_DEV - 2, subs(k), :],
                    va.at[r], copy_sems.at[r, 0])
                cp_b = pltpu.make_async_copy(
                    part_ref.at[sub_rows(o, k), cols(r)],
                    vb.at[r], copy_sems.at[r, 1])
                cp_a.start()
                cp_b.start()
                cp_a.wait()
                cp_b.wait()
                va[r] = _gelu(va[r] + vb[r])
                cp_o = pltpu.make_async_copy(
                    va.at[r], out_ref.at[sub_rows(o, k), cols(r)],
                    copy_sems.at[r, 2])
                cp_o.start()
                cp_o.wait()
                rd = pltpu.make_async_remote_copy(
                    src_ref=out_ref.at[sub_rows(o, k), cols(r)],
                    dst_ref=out_ref.at[sub_rows(o, k), cols(r)],
                    send_sem=ag_send_sems.at[r, 0, k],
                    recv_sem=ag_recv_sems.at[r, 0, k],
                    device_id=(peer(r),),
                    device_id_type=pl.DeviceIdType.MESH,
                )
                rd.start()
                end_waits.append(rd)

        for s in range(1, N_DEV - 1):
            for k in range(B):
                for r in (0, 1):
                    c_prev = ((d - s + 1) if r == 0 else (d + s - 1)) % N_DEV
                    rd = pltpu.make_async_remote_copy(
                        src_ref=out_ref.at[sub_rows(c_prev, k), cols(r)],
                        dst_ref=out_ref.at[sub_rows(c_prev, k), cols(r)],
                        send_sem=ag_send_sems.at[r, s - 1, k],
                        recv_sem=ag_recv_sems.at[r, s - 1, k],
                        device_id=(peer(r),),
                        device_id_type=pl.DeviceIdType.MESH,
                    )
                    rd.wait_recv()
                    fw = pltpu.make_async_remote_copy(
                        src_ref=out_ref.at[sub_rows(c_prev, k), cols(r)],
                        dst_ref=out_ref.at[sub_rows(c_prev, k), cols(r)],
                        send_sem=ag_send_sems.at[r, s, k],
                        recv_sem=ag_recv_sems.at[r, s, k],
                        device_id=(peer(r),),
                        device_id_type=pl.DeviceIdType.MESH,
                    )
                    fw.start()
                    end_waits.append(fw)

        for k in range(B):
            for r in (0, 1):
                c_last = ((d - N_DEV + 2) if r == 0 else (d + N_DEV - 2)) % N_DEV
                rd = pltpu.make_async_remote_copy(
                    src_ref=out_ref.at[sub_rows(c_last, k), cols(r)],
                    dst_ref=out_ref.at[sub_rows(c_last, k), cols(r)],
                    send_sem=ag_send_sems.at[r, N_DEV - 2, k],
                    recv_sem=ag_recv_sems.at[r, N_DEV - 2, k],
                    device_id=(peer(r),),
                    device_id_type=pl.DeviceIdType.MESH,
                )
                rd.wait_recv()
        for rd in end_waits:
            rd.wait_send()

    out = pl.pallas_call(
        body,
        out_shape=[
            jax.ShapeDtypeStruct((M, N), jnp.float32),
            jax.ShapeDtypeStruct((M, N), jnp.float32),
            jax.ShapeDtypeStruct((2, N_DEV - 1, CH, NH), jnp.float32),
            jax.ShapeDtypeStruct((2, N_DEV - 2, CH, NH), jnp.float32),
        ],
        in_specs=[pl.BlockSpec(memory_space=pl.ANY),
                  pl.BlockSpec(memory_space=pl.ANY)],
        out_specs=[pl.BlockSpec(memory_space=pl.ANY)] * 4,
        scratch_shapes=[
            pltpu.VMEM((2, KS, PW), jnp.float32),
            pltpu.VMEM((CH, KS), jnp.float32),
            pltpu.VMEM((2, SUB, PW), jnp.float32),
            pltpu.VMEM((2, SUB, NH), jnp.float32),
            pltpu.VMEM((2, SUB, NH), jnp.float32),
            pltpu.SemaphoreType.DMA((3, 2)),
            pltpu.SemaphoreType.DMA((2, 3)),
            pltpu.SemaphoreType.DMA((2, N_DEV - 1, B)),
            pltpu.SemaphoreType.DMA((2, N_DEV - 1, B)),
            pltpu.SemaphoreType.DMA((2, N_DEV - 1, B)),
            pltpu.SemaphoreType.DMA((2, N_DEV - 1, B)),
        ],
        compiler_params=pltpu.CompilerParams(collective_id=0),
    )(x, w_mat)[0]
    return out
